# baseline (speedup 1.0000x reference)
import jax
import jax.numpy as jnp
from jax import lax
from jax.experimental import pallas as pl
from jax.experimental.pallas import tpu as pltpu, tpu_sc as plsc

N_HEADS = 8
N_BOND = 5
N_DIST = 7

_info = plsc.get_sparse_core_info()
NC, NS, L = _info.num_cores, _info.num_subcores, _info.num_lanes
NW = NC * NS

EDGES = 512 * 128 * 128
E_PER_W = EDGES // NW
CHUNK = 4096
NCH = E_PER_W // CHUNK
OUT_CHUNK = CHUNK * N_HEADS
W = 128
CROWS = CHUNK // W
OROWS = OUT_CHUNK // W
OBLK = OROWS // 2


def _sc_body(bond_h, dist_h, btab_h, dtab_h, out_h,
             bond_v0, bond_v1, dist_v0, dist_v1, btab_v, dtab_v, tcomb_v,
             c8_v, out_v0, out_v1, iidx_v, oidx_v,
             sem_in0, sem_in1, sem_out0, sem_out1):
    wid = lax.axis_index("s") * NC + lax.axis_index("c")
    bond_v = (bond_v0, bond_v1)
    dist_v = (dist_v0, dist_v1)
    out_v = (out_v0, out_v1)
    sem_in = (sem_in0, sem_in1)
    sem_out = (sem_out0, sem_out1)

    iota = lax.iota(jnp.int32, L)
    h_off = iota & 7
    e_off = iota >> 3

    pltpu.sync_copy(btab_h, btab_v.at[pl.ds(0, N_BOND * N_HEADS)])
    pltpu.sync_copy(dtab_h, dtab_v.at[pl.ds(0, N_DIST * N_HEADS)])
    for k in range(18):
        t = iota + 16 * k
        c = t >> 3
        h = t & 7
        b = (c * 9363) >> 16
        d = c - b * 7
        vb = plsc.load_gather(btab_v, [b * 8 + h])
        vd = plsc.load_gather(dtab_v, [d * 8 + h])
        tcomb_v[pl.ds(16 * k, L)] = vb + vd

    def fill_iidx(g, p):
        row0 = (wid * E_PER_W + g * CHUNK) // W
        for j in range(CROWS // L):
            iidx_v[p, 0, pl.ds(16 * j, L)] = row0 + 16 * j + iota

    def fill_oidx(g, p):
        row0 = (wid * E_PER_W + g * CHUNK) * N_HEADS // W
        for k in range(2):
            for j in range(OBLK // L):
                oidx_v[p, k, pl.ds(16 * j, L)] = (
                    row0 + k * OBLK + 16 * j + iota)

    def in_dma(p):
        return (
            pltpu.make_async_copy(
                bond_h.at[iidx_v.at[p, 0]], bond_v[p], sem_in[p]),
            pltpu.make_async_copy(
                dist_h.at[iidx_v.at[p, 0]], dist_v[p], sem_in[p]),
        )

    def out_dma(p, k):
        return pltpu.make_async_copy(
            out_v[p].at[pl.ds(k * OBLK, OBLK)],
            out_h.at[oidx_v.at[p, k]], sem_out[p])

    for p in range(2):
        fill_iidx(p, p)
        for dsc in in_dma(p):
            dsc.start()

    def chunk(g, p):
        for dsc in in_dma(p):
            dsc.wait()

        bond_p, dist_p, out_p = bond_v[p], dist_v[p], out_v[p]

        @plsc.parallel_loop(0, CHUNK // L, unroll=8)
        def prep(j):
            r = j >> 3
            col = (j & 7) * 16
            b = bond_p[r, pl.ds(col, L)]
            d = dist_p[r, pl.ds(col, L)]
            c8_v[pl.ds(16 * j, L)] = (b * 7 + d) * 8

        @pl.when(g + 2 < NCH)
        def _():
            fill_iidx(g + 2, p)
            for dsc in in_dma(p):
                dsc.start()

        @pl.when(g >= 2)
        def _():
            for k in range(2):
                out_dma(p, k).wait()

        @plsc.parallel_loop(0, OUT_CHUNK // L, unroll=8)
        def main(i):
            e_vec = 2 * i + e_off
            a = plsc.load_gather(c8_v, [e_vec])
            v = plsc.load_gather(tcomb_v, [a + h_off])
            out_p[i >> 3, pl.ds((i & 7) * 16, L)] = v

        fill_oidx(g, p)
        for k in range(2):
            out_dma(p, k).start()

    def pair(go, carry):
        chunk(2 * go, 0)
        chunk(2 * go + 1, 1)
        return carry

    lax.fori_loop(0, NCH // 2, pair, 0)
    for p in range(2):
        for k in range(2):
            out_dma(p, k).wait()


@jax.jit
def _sc_call(bond_f, dist_f, btab, dtab):
    mesh = plsc.VectorSubcoreMesh(core_axis_name="c", subcore_axis_name="s")
    return pl.kernel(
        _sc_body,
        out_type=jax.ShapeDtypeStruct((EDGES * N_HEADS // W, W), jnp.float32),
        mesh=mesh,
        compiler_params=pltpu.CompilerParams(needs_layout_passes=False),
        scratch_types=[
            pltpu.VMEM((CROWS, W), jnp.int32),
            pltpu.VMEM((CROWS, W), jnp.int32),
            pltpu.VMEM((CROWS, W), jnp.int32),
            pltpu.VMEM((CROWS, W), jnp.int32),
            pltpu.VMEM((64,), jnp.float32),
            pltpu.VMEM((64,), jnp.float32),
            pltpu.VMEM((288,), jnp.float32),
            pltpu.VMEM((CHUNK,), jnp.int32),
            pltpu.VMEM((OROWS, W), jnp.float32),
            pltpu.VMEM((OROWS, W), jnp.float32),
            pltpu.VMEM((2, 1, CROWS), jnp.int32),
            pltpu.VMEM((2, 2, OBLK), jnp.int32),
            pltpu.SemaphoreType.DMA,
            pltpu.SemaphoreType.DMA,
            pltpu.SemaphoreType.DMA,
            pltpu.SemaphoreType.DMA,
        ],
    )(bond_f, dist_f, btab, dtab)


def kernel(bond_types, distances, bond_type_bias, distance_bias):
    shape = bond_types.shape
    out = _sc_call(
        bond_types.reshape(-1, W).astype(jnp.int32),
        distances.reshape(-1, W).astype(jnp.int32),
        bond_type_bias.reshape(-1).astype(jnp.float32),
        distance_bias.reshape(-1).astype(jnp.float32),
    )
    return out.reshape(*shape, N_HEADS)


# speedup vs baseline: 11.2184x; 11.2184x over previous
import jax
import jax.numpy as jnp
from jax import lax
from jax.experimental import pallas as pl
from jax.experimental.pallas import tpu as pltpu, tpu_sc as plsc

N_HEADS = 8
N_BOND = 5
N_DIST = 7

_info = plsc.get_sparse_core_info()
NC, NS, L = _info.num_cores, _info.num_subcores, _info.num_lanes
NW = NC * NS

EDGES = 512 * 128 * 128
E_PER_W = EDGES // NW
CHUNK = 4096
NCH = E_PER_W // CHUNK
OUT_CHUNK = CHUNK * N_HEADS
W = 128
CROWS = CHUNK // W
OROWS = OUT_CHUNK // W
OBLK = OROWS // 2


def _sc_body(bond_h, dist_h, btab_h, dtab_h, out_h,
             bond_v0, bond_v1, dist_v0, dist_v1, btab_v, dtab_v, tcomb_v,
             c8_v, out_v0, out_v1, iidx_v, oidx_v,
             sem_in0, sem_in1, sem_out0, sem_out1):
    wid = lax.axis_index("s") * NC + lax.axis_index("c")
    bond_v = (bond_v0, bond_v1)
    dist_v = (dist_v0, dist_v1)
    out_v = (out_v0, out_v1)
    sem_in = (sem_in0, sem_in1)
    sem_out = (sem_out0, sem_out1)

    iota = lax.iota(jnp.int32, L)

    pltpu.sync_copy(btab_h, btab_v.at[pl.ds(0, N_BOND * N_HEADS)])
    pltpu.sync_copy(dtab_h, dtab_v.at[pl.ds(0, N_DIST * N_HEADS)])
    for k in range(18):
        t = iota + 16 * k
        c = t >> 3
        h = t & 7
        b = (c * 9363) >> 16
        d = c - b * 7
        vb = plsc.load_gather(btab_v, [b * 8 + h])
        vd = plsc.load_gather(dtab_v, [d * 8 + h])
        tcomb_v[pl.ds(16 * k, L)] = vb + vd

    def fill_iidx(g, p):
        row0 = (wid * E_PER_W + g * CHUNK) // W
        for j in range(CROWS // L):
            iidx_v[p, 0, pl.ds(16 * j, L)] = row0 + 16 * j + iota

    def fill_oidx(g, p):
        row0 = (wid * E_PER_W + g * CHUNK) * N_HEADS // W
        for k in range(2):
            for j in range(OBLK // L):
                oidx_v[p, k, pl.ds(16 * j, L)] = (
                    row0 + k * OBLK + 16 * j + iota)

    def in_dma(p):
        return (
            pltpu.make_async_copy(
                bond_h.at[iidx_v.at[p, 0]], bond_v[p], sem_in[p]),
            pltpu.make_async_copy(
                dist_h.at[iidx_v.at[p, 0]], dist_v[p], sem_in[p]),
        )

    def out_dma(p, k):
        return pltpu.make_async_copy(
            out_v[p].at[pl.ds(k * OBLK, OBLK)],
            out_h.at[oidx_v.at[p, k]], sem_out[p])

    for p in range(2):
        fill_iidx(p, p)
        for dsc in in_dma(p):
            dsc.start()

    def chunk(g, p):
        for dsc in in_dma(p):
            dsc.wait()

        bond_p, dist_p, out_p = bond_v[p], dist_v[p], out_v[p]

        @plsc.parallel_loop(0, CHUNK // L, unroll=8)
        def prep(j):
            r = j >> 3
            col = (j & 7) * 16
            b = bond_p[r, pl.ds(col, L)]
            d = dist_p[r, pl.ds(col, L)]
            c8_v[pl.ds(16 * j, L)] = (b * 7 + d) * 8

        @pl.when(g + 2 < NCH)
        def _():
            fill_iidx(g + 2, p)
            for dsc in in_dma(p):
                dsc.start()

        @pl.when(g >= 2)
        def _():
            for k in range(2):
                out_dma(p, k).wait()

        @plsc.parallel_loop(0, CHUNK // L, unroll=4)
        def main(v):
            a = c8_v[pl.ds(16 * v, L)]
            r8 = (v >> 3) * 8
            col = (v & 7) * 16
            for h in range(N_HEADS):
                out_p[r8 + h, pl.ds(col, L)] = plsc.load_gather(
                    tcomb_v, [a + h])

        fill_oidx(g, p)
        for k in range(2):
            out_dma(p, k).start()

    def pair(go, carry):
        chunk(2 * go, 0)
        chunk(2 * go + 1, 1)
        return carry

    lax.fori_loop(0, NCH // 2, pair, 0)
    for p in range(2):
        for k in range(2):
            out_dma(p, k).wait()


@jax.jit
def _sc_call(bond_f, dist_f, btab, dtab):
    mesh = plsc.VectorSubcoreMesh(core_axis_name="c", subcore_axis_name="s")
    return pl.kernel(
        _sc_body,
        out_type=jax.ShapeDtypeStruct((EDGES * N_HEADS // W, W), jnp.float32),
        mesh=mesh,
        compiler_params=pltpu.CompilerParams(needs_layout_passes=False),
        scratch_types=[
            pltpu.VMEM((CROWS, W), jnp.int32),
            pltpu.VMEM((CROWS, W), jnp.int32),
            pltpu.VMEM((CROWS, W), jnp.int32),
            pltpu.VMEM((CROWS, W), jnp.int32),
            pltpu.VMEM((64,), jnp.float32),
            pltpu.VMEM((64,), jnp.float32),
            pltpu.VMEM((288,), jnp.float32),
            pltpu.VMEM((CHUNK,), jnp.int32),
            pltpu.VMEM((OROWS, W), jnp.float32),
            pltpu.VMEM((OROWS, W), jnp.float32),
            pltpu.VMEM((2, 1, CROWS), jnp.int32),
            pltpu.VMEM((2, 2, OBLK), jnp.int32),
            pltpu.SemaphoreType.DMA,
            pltpu.SemaphoreType.DMA,
            pltpu.SemaphoreType.DMA,
            pltpu.SemaphoreType.DMA,
        ],
    )(bond_f, dist_f, btab, dtab)


def kernel(bond_types, distances, bond_type_bias, distance_bias):
    shape = bond_types.shape
    out = _sc_call(
        bond_types.reshape(-1, W).astype(jnp.int32),
        distances.reshape(-1, W).astype(jnp.int32),
        bond_type_bias.reshape(-1).astype(jnp.float32),
        distance_bias.reshape(-1).astype(jnp.float32),
    )
    out = out.reshape(shape[0], shape[1], N_HEADS, shape[2])
    return jnp.transpose(out, (0, 1, 3, 2))


# speedup vs baseline: 22.6532x; 2.0193x over previous
import jax
import jax.numpy as jnp
from jax import lax
from jax.experimental import pallas as pl
from jax.experimental.pallas import tpu as pltpu, tpu_sc as plsc

N_HEADS = 8
N_BOND = 5
N_DIST = 7

_info = plsc.get_sparse_core_info()
NC, NS, L = _info.num_cores, _info.num_subcores, _info.num_lanes
NW = NC * NS

EDGES = 512 * 128 * 128
E_PER_W = EDGES // NW
CHUNK = 4096
NCH = E_PER_W // CHUNK
OUT_CHUNK = CHUNK * N_HEADS
W = 128
CROWS = CHUNK // W
OROWS = OUT_CHUNK // W
OBLK = OROWS // 2


def _sc_body(bond_h, dist_h, btab_h, dtab_h, out_h,
             bond_v0, bond_v1, dist_v0, dist_v1, btab_v, dtab_v, tcomb_v,
             trep_v, c8_v, out_v0, out_v1, iidx_v, oidx_v,
             sem_in0, sem_in1, sem_out0, sem_out1):
    wid = lax.axis_index("s") * NC + lax.axis_index("c")
    bond_v = (bond_v0, bond_v1)
    dist_v = (dist_v0, dist_v1)
    out_v = (out_v0, out_v1)
    sem_in = (sem_in0, sem_in1)
    sem_out = (sem_out0, sem_out1)

    iota = lax.iota(jnp.int32, L)

    pltpu.sync_copy(btab_h, btab_v.at[pl.ds(0, N_BOND * N_HEADS)])
    pltpu.sync_copy(dtab_h, dtab_v.at[pl.ds(0, N_DIST * N_HEADS)])
    for k in range(18):
        t = iota + 16 * k
        c = t >> 3
        h = t & 7
        b = (c * 9363) >> 16
        d = c - b * 7
        vb = plsc.load_gather(btab_v, [b * 8 + h])
        vd = plsc.load_gather(dtab_v, [d * 8 + h])
        tcomb_v[pl.ds(16 * k, L)] = vb + vd

    def repl(t, carry):
        v = tcomb_v[pl.ds(t, L)]
        trep_v[pl.ds(16 * t, L)] = jnp.broadcast_to(v[0], (L,))
        return carry

    lax.fori_loop(0, 280, repl, 0)

    def fill_iidx(g, p):
        row0 = (wid * E_PER_W + g * CHUNK) // W
        for j in range(CROWS // L):
            iidx_v[p, 0, pl.ds(16 * j, L)] = row0 + 16 * j + iota

    def fill_oidx(g, p):
        row0 = (wid * E_PER_W + g * CHUNK) * N_HEADS // W
        for k in range(2):
            for j in range(OBLK // L):
                oidx_v[p, k, pl.ds(16 * j, L)] = (
                    row0 + k * OBLK + 16 * j + iota)

    def in_dma(p):
        return (
            pltpu.make_async_copy(
                bond_h.at[iidx_v.at[p, 0]], bond_v[p], sem_in[p]),
            pltpu.make_async_copy(
                dist_h.at[iidx_v.at[p, 0]], dist_v[p], sem_in[p]),
        )

    def out_dma(p, k):
        return pltpu.make_async_copy(
            out_v[p].at[pl.ds(k * OBLK, OBLK)],
            out_h.at[oidx_v.at[p, k]], sem_out[p])

    for p in range(2):
        fill_iidx(p, p)
        for dsc in in_dma(p):
            dsc.start()

    def chunk(g, p):
        for dsc in in_dma(p):
            dsc.wait()

        bond_p, dist_p, out_p = bond_v[p], dist_v[p], out_v[p]

        @plsc.parallel_loop(0, CHUNK // L, unroll=8)
        def prep(j):
            r = j >> 3
            col = (j & 7) * 16
            b = bond_p[r, pl.ds(col, L)]
            d = dist_p[r, pl.ds(col, L)]
            c8_v[pl.ds(16 * j, L)] = (b * 7 + d) * 128

        @pl.when(g + 2 < NCH)
        def _():
            fill_iidx(g + 2, p)
            for dsc in in_dma(p):
                dsc.start()

        @pl.when(g >= 2)
        def _():
            for k in range(2):
                out_dma(p, k).wait()

        @plsc.parallel_loop(0, CHUNK // L, unroll=4)
        def main(v):
            a = c8_v[pl.ds(16 * v, L)]
            r8 = (v >> 3) * 8
            col = (v & 7) * 16
            for h in range(N_HEADS):
                out_p[r8 + h, pl.ds(col, L)] = plsc.load_gather(
                    trep_v, [a + (16 * h) + iota])

        fill_oidx(g, p)
        for k in range(2):
            out_dma(p, k).start()

    def pair(go, carry):
        chunk(2 * go, 0)
        chunk(2 * go + 1, 1)
        return carry

    lax.fori_loop(0, NCH // 2, pair, 0)
    for p in range(2):
        for k in range(2):
            out_dma(p, k).wait()


@jax.jit
def _sc_call(bond_f, dist_f, btab, dtab):
    mesh = plsc.VectorSubcoreMesh(core_axis_name="c", subcore_axis_name="s")
    return pl.kernel(
        _sc_body,
        out_type=jax.ShapeDtypeStruct((EDGES * N_HEADS // W, W), jnp.float32),
        mesh=mesh,
        compiler_params=pltpu.CompilerParams(needs_layout_passes=False),
        scratch_types=[
            pltpu.VMEM((CROWS, W), jnp.int32),
            pltpu.VMEM((CROWS, W), jnp.int32),
            pltpu.VMEM((CROWS, W), jnp.int32),
            pltpu.VMEM((CROWS, W), jnp.int32),
            pltpu.VMEM((64,), jnp.float32),
            pltpu.VMEM((64,), jnp.float32),
            pltpu.VMEM((304,), jnp.float32),
            pltpu.VMEM((4480,), jnp.float32),
            pltpu.VMEM((CHUNK,), jnp.int32),
            pltpu.VMEM((OROWS, W), jnp.float32),
            pltpu.VMEM((OROWS, W), jnp.float32),
            pltpu.VMEM((2, 1, CROWS), jnp.int32),
            pltpu.VMEM((2, 2, OBLK), jnp.int32),
            pltpu.SemaphoreType.DMA,
            pltpu.SemaphoreType.DMA,
            pltpu.SemaphoreType.DMA,
            pltpu.SemaphoreType.DMA,
        ],
    )(bond_f, dist_f, btab, dtab)


def kernel(bond_types, distances, bond_type_bias, distance_bias):
    shape = bond_types.shape
    out = _sc_call(
        bond_types.reshape(-1, W).astype(jnp.int32),
        distances.reshape(-1, W).astype(jnp.int32),
        bond_type_bias.reshape(-1).astype(jnp.float32),
        distance_bias.reshape(-1).astype(jnp.float32),
    )
    out = out.reshape(shape[0], shape[1], N_HEADS, shape[2])
    return jnp.transpose(out, (0, 1, 3, 2))
